# baseline (speedup 1.0000x reference)
import functools

import jax
import jax.numpy as jnp
from jax import lax
from jax.experimental import pallas as pl
from jax.experimental.pallas import tpu as pltpu
from jax.experimental.pallas import tpu_sc as plsc

N = 10000
E = 320000
D = 128

NC = 2
NS = 16
NW = NC * NS
EP = E // NW
CB = 80
NCHUNK = EP // CB
GC = 25
NG = NCHUNK // GC
NBUF = 4
RPT = 624
TAIL = N - NS * RPT

_mesh = plsc.VectorSubcoreMesh(core_axis_name="c", subcore_axis_name="s")


@functools.partial(
    pl.kernel,
    out_type=jax.ShapeDtypeStruct((NW, N), jnp.float32),
    mesh=_mesh,
    compiler_params=pltpu.CompilerParams(needs_layout_passes=False),
    scratch_types=[
        pltpu.VMEM((GC, CB), jnp.int32),
        pltpu.VMEM((N,), jnp.float32),
    ],
)
def _deg_kernel(ei_hbm, out_hbm, idx_v, hist_v):
    c = lax.axis_index("c")
    s = lax.axis_index("s")
    wid = s * NC + c

    zeros16 = jnp.zeros((16,), jnp.float32)

    @pl.loop(0, N // 16)
    def _(i):
        hist_v[pl.ds(i * 16, 16)] = zeros16

    ones16 = jnp.ones((16,), jnp.float32)

    @pl.loop(0, NG)
    def _(g):
        pltpu.sync_copy(ei_hbm.at[1, wid, g], idx_v)

        @pl.loop(0, GC)
        def _(r):
            for c16 in range(CB // 16):
                idx = idx_v[r, pl.ds(c16 * 16, 16)]
                plsc.addupdate_scatter(hist_v, [idx], ones16)

    pltpu.sync_copy(hist_v, out_hbm.at[wid])


@functools.partial(
    pl.kernel,
    out_type=jax.ShapeDtypeStruct((NC, N, D), jnp.float32),
    mesh=_mesh,
    scratch_types=[
        pltpu.VMEM((GC, CB), jnp.int32),
        pltpu.VMEM((GC, CB), jnp.int32),
        pltpu.VMEM((NBUF, CB, D), jnp.float32),
        pltpu.VMEM_SHARED((N, D), jnp.float32),
        pltpu.SemaphoreType.DMA,
        pltpu.SemaphoreType.DMA,
    ],
)
def _edge_kernel(y_hbm, ei_hbm, out_hbm,
                 src_v, dst_v, rows_v, acc_s, gsem, ssem):
    c = lax.axis_index("c")
    s = lax.axis_index("s")
    wid = s * NC + c

    z16 = jnp.zeros((16,), jnp.float32)

    @pl.loop(0, CB)
    def _(r):
        for c16 in range(D // 16):
            rows_v[0, r, pl.ds(c16 * 16, 16)] = z16

    for k in range(RPT // CB):
        pltpu.sync_copy(rows_v.at[0], acc_s.at[pl.ds(s * RPT + k * CB, CB)])
    _rem = RPT - (RPT // CB) * CB
    pltpu.sync_copy(rows_v.at[0, pl.ds(0, _rem)],
                    acc_s.at[pl.ds(s * RPT + RPT - _rem, _rem)])

    @pl.when(s == NS - 1)
    def _():
        pltpu.sync_copy(rows_v.at[0, pl.ds(0, TAIL)],
                        acc_s.at[pl.ds(NS * RPT, TAIL)])

    plsc.subcore_barrier()

    @pl.loop(0, NG)
    def _(g):
        pltpu.sync_copy(ei_hbm.at[0, wid, g], src_v)
        pltpu.sync_copy(ei_hbm.at[1, wid, g], dst_v)
        for p in range(NBUF - 1):
            pltpu.async_copy(y_hbm.at[src_v.at[p]], rows_v.at[p], gsem)

        @pl.loop(0, GC)
        def _(i):
            nxt = i + NBUF - 1

            @pl.when(nxt < GC)
            def _():
                @pl.when(i >= 1)
                def _():
                    pltpu.make_async_copy(
                        rows_v.at[0], acc_s.at[dst_v.at[0]], ssem).wait()

                pltpu.async_copy(y_hbm.at[src_v.at[nxt]],
                                 rows_v.at[nxt % NBUF], gsem)

            pltpu.make_async_copy(y_hbm.at[src_v.at[i]],
                                  rows_v.at[i % NBUF], gsem).wait()
            pltpu.async_copy(rows_v.at[i % NBUF], acc_s.at[dst_v.at[i]],
                             ssem, add=True)

        for p in range(NBUF):
            pltpu.make_async_copy(rows_v.at[0], acc_s.at[dst_v.at[0]],
                                  ssem).wait()

    plsc.subcore_barrier()
    pltpu.sync_copy(acc_s.at[pl.ds(s * RPT, RPT)],
                    out_hbm.at[c, pl.ds(s * RPT, RPT)])

    @pl.when(s == NS - 1)
    def _():
        pltpu.sync_copy(acc_s.at[pl.ds(NS * RPT, TAIL)],
                        out_hbm.at[c, pl.ds(NS * RPT, TAIL)])


def _dinv_from_parts(parts):
    deg = jnp.sum(parts, axis=0) + 1.0
    return lax.rsqrt(jnp.maximum(deg, 1.0))


def _pre_body(x_ref, w_ref, parts_ref, y_ref):
    dinv = _dinv_from_parts(parts_ref[...])
    xw = jnp.dot(x_ref[...], w_ref[...], preferred_element_type=jnp.float32)
    y_ref[...] = xw * dinv[:, None]


def _mid_body(acc_ref, y_ref, parts_ref, b_ref, w_ref, y2_ref):
    dinv = _dinv_from_parts(parts_ref[...])[:, None]
    h = (acc_ref[0] + acc_ref[1] + y_ref[...]) * dinv + b_ref[...]
    h = jnp.maximum(h, 0.0)
    y2_ref[...] = jnp.dot(h, w_ref[...],
                          preferred_element_type=jnp.float32) * dinv


def _post_body(acc_ref, y_ref, parts_ref, b_ref, out_ref):
    dinv = _dinv_from_parts(parts_ref[...])[:, None]
    out_ref[...] = (acc_ref[0] + acc_ref[1] + y_ref[...]) * dinv + b_ref[...]


_f32 = jnp.float32
_tc_pre = pl.pallas_call(
    _pre_body, out_shape=jax.ShapeDtypeStruct((N, D), _f32))
_tc_mid = pl.pallas_call(
    _mid_body, out_shape=jax.ShapeDtypeStruct((N, D), _f32))
_tc_post = pl.pallas_call(
    _post_body, out_shape=jax.ShapeDtypeStruct((N, D), _f32))


def kernel(x, edge_index, W1, b1, W2, b2):
    ei4 = edge_index.reshape(2, NW, NG, GC, CB)

    parts = _deg_kernel(ei4)
    y1 = _tc_pre(x, W1, parts)
    acc1 = _edge_kernel(y1, ei4)
    y2 = _tc_mid(acc1, y1, parts, b1.reshape(1, D), W2)
    acc2 = _edge_kernel(y2, ei4)
    return _tc_post(acc2, y2, parts, b2.reshape(1, D))


# speedup vs baseline: 1.0263x; 1.0263x over previous
import functools

import jax
import jax.numpy as jnp
from jax import lax
from jax.experimental import pallas as pl
from jax.experimental.pallas import tpu as pltpu
from jax.experimental.pallas import tpu_sc as plsc

N = 10000
E = 320000
D = 128

NC = 2
NS = 16
NW = NC * NS
EP = E // NW
CB = 80
NCHUNK = EP // CB
GC = 25
NG = NCHUNK // GC
NBUF = 3
RPT = 624
TAIL = N - NS * RPT

_mesh = plsc.VectorSubcoreMesh(core_axis_name="c", subcore_axis_name="s")


@functools.partial(
    pl.kernel,
    out_type=jax.ShapeDtypeStruct((NW, N), jnp.float32),
    mesh=_mesh,
    compiler_params=pltpu.CompilerParams(needs_layout_passes=False),
    scratch_types=[
        pltpu.VMEM((EP,), jnp.int32),
        pltpu.VMEM((N,), jnp.float32),
    ],
)
def _deg_kernel(ei_hbm, out_hbm, idx_v, hist_v):
    c = lax.axis_index("c")
    s = lax.axis_index("s")
    wid = s * NC + c
    pltpu.sync_copy(ei_hbm.at[1, wid], idx_v)

    zeros16 = jnp.zeros((16,), jnp.float32)

    @pl.loop(0, N // 16)
    def _(i):
        hist_v[pl.ds(i * 16, 16)] = zeros16

    ones16 = jnp.ones((16,), jnp.float32)

    @pl.loop(0, EP // 16)
    def _(j):
        idx = idx_v[pl.ds(j * 16, 16)]
        plsc.addupdate_scatter(hist_v, [idx], ones16)

    pltpu.sync_copy(hist_v, out_hbm.at[wid])


@functools.partial(
    pl.kernel,
    out_type=jax.ShapeDtypeStruct((NC, N, D), jnp.float32),
    mesh=_mesh,
    scratch_types=[
        pltpu.VMEM((2, GC, CB), jnp.int32),
        pltpu.VMEM((2, GC, CB), jnp.int32),
        pltpu.VMEM((NBUF, CB, D), jnp.float32),
        pltpu.VMEM_SHARED((N, D), jnp.float32),
        pltpu.SemaphoreType.DMA,
        pltpu.SemaphoreType.DMA,
        pltpu.SemaphoreType.DMA,
    ],
)
def _edge_kernel(y_hbm, ei_hbm, out_hbm,
                 src_v, dst_v, rows_v, acc_s, gsem, ssem, isem):
    c = lax.axis_index("c")
    s = lax.axis_index("s")
    wid = s * NC + c

    z16 = jnp.zeros((16,), jnp.float32)

    @pl.loop(0, CB)
    def _(r):
        for c16 in range(D // 16):
            rows_v[0, r, pl.ds(c16 * 16, 16)] = z16

    for k in range(RPT // CB):
        pltpu.sync_copy(rows_v.at[0], acc_s.at[pl.ds(s * RPT + k * CB, CB)])
    _rem = RPT - (RPT // CB) * CB
    pltpu.sync_copy(rows_v.at[0, pl.ds(0, _rem)],
                    acc_s.at[pl.ds(s * RPT + RPT - _rem, _rem)])

    @pl.when(s == NS - 1)
    def _():
        pltpu.sync_copy(rows_v.at[0, pl.ds(0, TAIL)],
                        acc_s.at[pl.ds(NS * RPT, TAIL)])

    plsc.subcore_barrier()

    pltpu.async_copy(ei_hbm.at[0, wid, 0], src_v.at[0], isem)
    pltpu.async_copy(ei_hbm.at[1, wid, 0], dst_v.at[0], isem)

    @pl.loop(0, NG)
    def _(g):
        b = g % 2
        pltpu.make_async_copy(ei_hbm.at[0, wid, 0], src_v.at[0], isem).wait()
        pltpu.make_async_copy(ei_hbm.at[1, wid, 0], dst_v.at[0], isem).wait()

        @pl.when(g + 1 < NG)
        def _():
            pltpu.async_copy(ei_hbm.at[0, wid, g + 1],
                             src_v.at[1 - b], isem)
            pltpu.async_copy(ei_hbm.at[1, wid, g + 1],
                             dst_v.at[1 - b], isem)

        for p in range(NBUF - 1):
            pltpu.async_copy(y_hbm.at[src_v.at[b, p]], rows_v.at[p], gsem)

        @pl.loop(0, GC)
        def _(i):
            nxt = i + NBUF - 1

            @pl.when(nxt < GC)
            def _():
                @pl.when(i >= 1)
                def _():
                    pltpu.make_async_copy(
                        rows_v.at[0], acc_s.at[dst_v.at[0, 0]],
                        ssem).wait()

                pltpu.async_copy(y_hbm.at[src_v.at[b, nxt]],
                                 rows_v.at[nxt % NBUF], gsem)

            pltpu.make_async_copy(y_hbm.at[src_v.at[b, i]],
                                  rows_v.at[i % NBUF], gsem).wait()
            pltpu.async_copy(rows_v.at[i % NBUF], acc_s.at[dst_v.at[b, i]],
                             ssem, add=True)

        for p in range(NBUF):
            pltpu.make_async_copy(rows_v.at[0], acc_s.at[dst_v.at[0, 0]],
                                  ssem).wait()

    plsc.subcore_barrier()
    pltpu.sync_copy(acc_s.at[pl.ds(s * RPT, RPT)],
                    out_hbm.at[c, pl.ds(s * RPT, RPT)])

    @pl.when(s == NS - 1)
    def _():
        pltpu.sync_copy(acc_s.at[pl.ds(NS * RPT, TAIL)],
                        out_hbm.at[c, pl.ds(NS * RPT, TAIL)])


def _dinv_from_parts(parts):
    deg = jnp.sum(parts, axis=0) + 1.0
    return lax.rsqrt(jnp.maximum(deg, 1.0))


def _pre_body(x_ref, w_ref, parts_ref, y_ref):
    dinv = _dinv_from_parts(parts_ref[...])
    xw = jnp.dot(x_ref[...], w_ref[...], preferred_element_type=jnp.float32)
    y_ref[...] = xw * dinv[:, None]


def _mid_body(acc_ref, y_ref, parts_ref, b_ref, w_ref, y2_ref):
    dinv = _dinv_from_parts(parts_ref[...])[:, None]
    h = (acc_ref[0] + acc_ref[1] + y_ref[...]) * dinv + b_ref[...]
    h = jnp.maximum(h, 0.0)
    y2_ref[...] = jnp.dot(h, w_ref[...],
                          preferred_element_type=jnp.float32) * dinv


def _post_body(acc_ref, y_ref, parts_ref, b_ref, out_ref):
    dinv = _dinv_from_parts(parts_ref[...])[:, None]
    out_ref[...] = (acc_ref[0] + acc_ref[1] + y_ref[...]) * dinv + b_ref[...]


_f32 = jnp.float32
_tc_pre = pl.pallas_call(
    _pre_body, out_shape=jax.ShapeDtypeStruct((N, D), _f32))
_tc_mid = pl.pallas_call(
    _mid_body, out_shape=jax.ShapeDtypeStruct((N, D), _f32))
_tc_post = pl.pallas_call(
    _post_body, out_shape=jax.ShapeDtypeStruct((N, D), _f32))


def kernel(x, edge_index, W1, b1, W2, b2):
    ei4 = edge_index.reshape(2, NW, NG, GC, CB)
    ei_flat = edge_index.reshape(2, NW, EP)

    parts = _deg_kernel(ei_flat)
    y1 = _tc_pre(x, W1, parts)
    acc1 = _edge_kernel(y1, ei4)
    y2 = _tc_mid(acc1, y1, parts, b1.reshape(1, D), W2)
    acc2 = _edge_kernel(y2, ei4)
    return _tc_post(acc2, y2, parts, b2.reshape(1, D))
